# baseline (speedup 1.0000x reference)
import jax
import jax.numpy as jnp
from jax import lax
from jax.experimental import pallas as pl
from jax.experimental.pallas import tpu as pltpu
from jax.experimental.pallas import tpu_sc as plsc

_USER = 5000
_N = 10000
_NP = 10240
_DH = 64
_E = 320000
_LANES = 128
_RPW = 1
_WIN = 160
_RPS = _WIN * _RPW
_EPAD = 16 * _RPS * _LANES
_EROWS = _EPAD // _LANES
_NROW = _NP // 16


def _pack_edges(index, vals):
    pad = _EPAD - _E
    ar = jnp.arange(pad, dtype=jnp.int32)
    rows = jnp.concatenate([index[0], ar % _N]).reshape(_EROWS, _LANES)
    cols = jnp.concatenate([index[1], (ar * 7919) % _N]).reshape(_EROWS, _LANES)
    v = jnp.concatenate([vals, jnp.zeros((pad,), vals.dtype)])
    return jnp.stack([rows, cols], axis=1), v


def _spmm_pass(sid, coff, edg_hbm, val_hbm, src, dst_sh,
               ebuf, vbuf, gath, sbuf,
               esem0, esem1, gsem0, gsem1, ssem0, ssem1, off):
    esems = (esem0, esem1)
    gsems = (gsem0, gsem1)
    ssems = (ssem0, ssem1)

    def fire_edges(w, s, p):
        base = sid * _RPS + w * _RPW
        pltpu.async_copy(edg_hbm.at[pl.ds(base, _RPW)], ebuf.at[s], esems[p])
        pltpu.async_copy(val_hbm.at[pl.ds(base * _LANES, _RPW * _LANES)],
                         vbuf.at[s], esems[p])

    def drain_edges(s, p):
        pltpu.make_async_copy(edg_hbm.at[pl.ds(0, _RPW)], ebuf.at[s],
                              esems[p]).wait()
        pltpu.make_async_copy(val_hbm.at[pl.ds(0, _RPW * _LANES)],
                              vbuf.at[s], esems[p]).wait()

    def offset_cols(s):
        @plsc.parallel_loop(0, _RPW * (_LANES // 16), unroll=2)
        def oi(i):
            j = i // (_LANES // 16)
            g = i % (_LANES // 16)
            sl = (s, j, 1, pl.ds(g * 16, 16))
            ebuf[sl] = ebuf[sl] + coff

    def fire_gathers(b, s):
        for j in range(_RPW):
            pltpu.async_copy(src.at[ebuf.at[s, j, 1]], gath.at[b, j],
                             gsems[b])

    def drain_gathers(b):
        for j in range(_RPW):
            pltpu.make_async_copy(src.at[pl.ds(0, _LANES)],
                                  gath.at[b, j], gsems[b]).wait()

    def scale(b, s):
        def sj(j, c):
            @plsc.parallel_loop(0, _LANES // 16, unroll=4)
            def sg(g):
                vv = vbuf[s, pl.ds(j * _LANES + g * 16, 16)]
                for e2 in range(16):
                    vf = jnp.broadcast_to(vv[e2], (16,))
                    e = g * 16 + e2
                    for k in range(_DH // 16):
                        sl = (b, j, e, pl.ds(k * 16, 16))
                        sbuf[sl] = gath[sl] * vf
            return c
        lax.fori_loop(0, _RPW, sj, 0)

    def fire_scatters(b, s):
        for j in range(_RPW):
            pltpu.async_copy(sbuf.at[b, j], dst_sh.at[ebuf.at[s, j, 0]],
                             ssems[b], add=True)

    def drain_scatters(b):
        for j in range(_RPW):
            pltpu.make_async_copy(sbuf.at[b, j], dst_sh.at[pl.ds(0, _LANES)],
                                  ssems[b]).wait()

    def nothing():
        pass

    fire_edges(0, 0, 0)
    fire_edges(1, 1, 1)
    drain_edges(0, 0)
    if off:
        offset_cols(0)
    fire_gathers(0, 0)

    def pair(i, carry):
        su = 2 * lax.rem(i, 2)
        for u in (0, 1):
            w = 2 * i + u
            b = u
            s_w = su + u
            s_w1 = su + 1 if u == 0 else 2 - su
            s_w2 = 2 - su if u == 0 else 3 - su

            def prep_next(s1=s_w1, b1=1 - b):
                drain_edges(s1, b1 % 2)
                if off:
                    offset_cols(s1)
                fire_gathers(b1, s1)

            lax.cond(w < _WIN - 1, prep_next, nothing)
            drain_gathers(b)
            lax.cond(w >= 2, lambda bb=b: drain_scatters(bb), nothing)
            lax.cond(w <= _WIN - 3,
                     lambda ww=w, ss=s_w2, bb=b: fire_edges(ww + 2, ss, bb),
                     nothing)
            scale(b, s_w)
            fire_scatters(b, s_w)
        return carry

    lax.fori_loop(0, _WIN // 2, pair, 0)
    drain_scatters(0)
    drain_scatters(1)


def _body(edges_a, vals_a, edges_d, vals_d, xcat,
          mean_out, cond_out,
          acc, acc2, ebuf, vbuf, gath, sbuf,
          esem0, esem1, gsem0, gsem1, ssem0, ssem1):
    cid = lax.axis_index("c")
    sid = lax.axis_index("s")
    coff = cid * _NP
    rbase = sid * _NROW

    def fill_zero_chunk():
        @plsc.parallel_loop(0, _LANES, unroll=4)
        def zr(r):
            for k in range(_DH // 16):
                sbuf[(0, 0, r, pl.ds(k * 16, 16))] = jnp.zeros((16,),
                                                               jnp.float32)

    def zero_slice(dst):
        for t in range(_NROW // _LANES):
            pltpu.sync_copy(sbuf.at[0, 0],
                            dst.at[pl.ds(rbase + t * _LANES, _LANES)])

    fill_zero_chunk()
    zero_slice(acc)
    zero_slice(acc2)
    plsc.subcore_barrier()

    _spmm_pass(sid, coff, edges_a, vals_a, xcat, acc,
               ebuf, vbuf, gath, sbuf, esem0, esem1, gsem0, gsem1,
               ssem0, ssem1, True)
    plsc.subcore_barrier()

    _spmm_pass(sid, coff, edges_a, vals_a, acc, acc2,
               ebuf, vbuf, gath, sbuf, esem0, esem1, gsem0, gsem1,
               ssem0, ssem1, False)
    plsc.subcore_barrier()

    bx, bh, b2 = gath.at[0, 0], gath.at[0, 1], sbuf.at[0, 0]
    for t in range(5):
        r0 = rbase + t * 128
        pltpu.sync_copy(xcat.at[pl.ds(coff + r0, 128)], bx)
        pltpu.sync_copy(acc.at[pl.ds(r0, 128)], bh)
        pltpu.sync_copy(acc2.at[pl.ds(r0, 128)], b2)

        @plsc.parallel_loop(0, 128, unroll=4)
        def mrow(r):
            for k in range(_DH // 16):
                sl = (r, pl.ds(k * 16, 16))
                bx[sl] = (bx[sl] + bh[sl] + b2[sl]) * jnp.float32(1.0 / 3.0)
        pltpu.sync_copy(bx, mean_out.at[pl.ds(coff + r0, 128)])
        pltpu.sync_copy(bx, acc.at[pl.ds(r0, 128)])
    fill_zero_chunk()
    zero_slice(acc2)
    plsc.subcore_barrier()

    _spmm_pass(sid, coff, edges_d, vals_d, acc, acc2,
               ebuf, vbuf, gath, sbuf, esem0, esem1, gsem0, gsem1,
               ssem0, ssem1, False)
    plsc.subcore_barrier()
    pltpu.sync_copy(acc2.at[pl.ds(rbase, _NROW)],
                    cond_out.at[pl.ds(coff + rbase, _NROW)])


def kernel(adj_index, adj_vals, d1_index, d1_vals, uEmbeds, iEmbeds):
    embeds = jnp.concatenate([uEmbeds, iEmbeds], axis=0)
    zpad = jnp.zeros((_NP - _N, _DH), jnp.float32)
    xcat = jnp.concatenate(
        [embeds[:, :_DH], zpad, embeds[:, _DH:], zpad], axis=0)
    edges_a, vals_a = _pack_edges(adj_index, adj_vals)
    edges_d, vals_d = _pack_edges(d1_index, d1_vals)

    mesh = plsc.VectorSubcoreMesh(core_axis_name="c", subcore_axis_name="s",
                                  num_cores=2, num_subcores=16)
    f32 = jnp.float32
    half = jax.ShapeDtypeStruct((2 * _NP, _DH), f32)
    call = pl.kernel(
        _body,
        out_type=(half, half),
        mesh=mesh,
        compiler_params=pltpu.CompilerParams(use_tc_tiling_on_sc=False),
        scratch_types=[
            pltpu.VMEM_SHARED((_NP, _DH), f32),
            pltpu.VMEM_SHARED((_NP, _DH), f32),
            pltpu.VMEM((4, _RPW, 2, _LANES), jnp.int32),
            pltpu.VMEM((4, _RPW * _LANES), f32),
            pltpu.VMEM((2, _RPW, _LANES, _DH), f32),
            pltpu.VMEM((2, _RPW, _LANES, _DH), f32),
            pltpu.SemaphoreType.DMA,
            pltpu.SemaphoreType.DMA,
            pltpu.SemaphoreType.DMA,
            pltpu.SemaphoreType.DMA,
            pltpu.SemaphoreType.DMA,
            pltpu.SemaphoreType.DMA,
        ],
    )
    mean_h, cond_h = call(edges_a, vals_a, edges_d, vals_d, xcat)
    mean = jnp.concatenate([mean_h[:_N], mean_h[_NP:_NP + _N]], axis=1)
    cond = jnp.concatenate([cond_h[:_N], cond_h[_NP:_NP + _N]], axis=1)
    return mean[:_USER], mean[_USER:], cond, uEmbeds, iEmbeds


# speedup vs baseline: 1.0749x; 1.0749x over previous
import jax
import jax.numpy as jnp
from jax import lax
from jax.experimental import pallas as pl
from jax.experimental.pallas import tpu as pltpu
from jax.experimental.pallas import tpu_sc as plsc

_USER = 5000
_N = 10000
_NP = 10240
_DH = 64
_E = 320000
_LANES = 128
_RPW = 2
_WIN = 80
_RPS = _WIN * _RPW
_EPAD = 16 * _RPS * _LANES
_EROWS = _EPAD // _LANES
_NROW = _NP // 16


def _pack_edges(index, vals):
    pad = _EPAD - _E
    ar = jnp.arange(pad, dtype=jnp.int32)
    rows = jnp.concatenate([index[0], ar % _N]).reshape(_EROWS, _LANES)
    cols = jnp.concatenate([index[1], (ar * 7919) % _N]).reshape(_EROWS, _LANES)
    v = jnp.concatenate([vals, jnp.zeros((pad,), vals.dtype)])
    return jnp.stack([rows, cols], axis=1), v


def _spmm_pass(sid, coff, edg_hbm, val_hbm, src, dst_sh,
               ebuf, vbuf, gath, sbuf,
               esem0, esem1, gsem0, gsem1, ssem0, ssem1, off):
    esems = (esem0, esem1)
    gsems = (gsem0, gsem1)
    ssems = (ssem0, ssem1)

    def fire_edges(w, s, p):
        base = sid * _RPS + w * _RPW
        pltpu.async_copy(edg_hbm.at[pl.ds(base, _RPW)], ebuf.at[s], esems[p])
        pltpu.async_copy(val_hbm.at[pl.ds(base * _LANES, _RPW * _LANES)],
                         vbuf.at[s], esems[p])

    def drain_edges(s, p):
        pltpu.make_async_copy(edg_hbm.at[pl.ds(0, _RPW)], ebuf.at[s],
                              esems[p]).wait()
        pltpu.make_async_copy(val_hbm.at[pl.ds(0, _RPW * _LANES)],
                              vbuf.at[s], esems[p]).wait()

    def offset_cols(s):
        @plsc.parallel_loop(0, _RPW * (_LANES // 16), unroll=2)
        def oi(i):
            j = i // (_LANES // 16)
            g = i % (_LANES // 16)
            sl = (s, j, 1, pl.ds(g * 16, 16))
            ebuf[sl] = ebuf[sl] + coff

    def fire_gathers(b, s):
        for j in range(_RPW):
            pltpu.async_copy(src.at[ebuf.at[s, j, 1]], gath.at[b, j],
                             gsems[b])

    def drain_gathers(b):
        for j in range(_RPW):
            pltpu.make_async_copy(src.at[pl.ds(0, _LANES)],
                                  gath.at[b, j], gsems[b]).wait()

    def scale(b, s):
        def sj(j, c):
            @plsc.parallel_loop(0, _LANES // 16, unroll=4)
            def sg(g):
                vv = vbuf[s, pl.ds(j * _LANES + g * 16, 16)]
                for e2 in range(16):
                    vf = jnp.broadcast_to(vv[e2], (16,))
                    e = g * 16 + e2
                    for k in range(_DH // 16):
                        sl = (b, j, e, pl.ds(k * 16, 16))
                        sbuf[sl] = gath[sl] * vf
            return c
        lax.fori_loop(0, _RPW, sj, 0)

    def fire_scatters(b, s):
        for j in range(_RPW):
            pltpu.async_copy(sbuf.at[b, j], dst_sh.at[ebuf.at[s, j, 0]],
                             ssems[b], add=True)

    def drain_scatters(b):
        for j in range(_RPW):
            pltpu.make_async_copy(sbuf.at[b, j], dst_sh.at[pl.ds(0, _LANES)],
                                  ssems[b]).wait()

    def nothing():
        pass

    fire_edges(0, 0, 0)
    fire_edges(1, 1, 1)
    drain_edges(0, 0)
    if off:
        offset_cols(0)
    fire_gathers(0, 0)

    def pair(i, carry):
        su = 2 * lax.rem(i, 2)
        for u in (0, 1):
            w = 2 * i + u
            b = u
            s_w = su + u
            s_w1 = su + 1 if u == 0 else 2 - su
            s_w2 = 2 - su if u == 0 else 3 - su

            def prep_next(s1=s_w1, b1=1 - b):
                drain_edges(s1, b1 % 2)
                if off:
                    offset_cols(s1)
                fire_gathers(b1, s1)

            lax.cond(w < _WIN - 1, prep_next, nothing)
            drain_gathers(b)
            lax.cond(w >= 2, lambda bb=b: drain_scatters(bb), nothing)
            lax.cond(w <= _WIN - 3,
                     lambda ww=w, ss=s_w2, bb=b: fire_edges(ww + 2, ss, bb),
                     nothing)
            scale(b, s_w)
            fire_scatters(b, s_w)
        return carry

    lax.fori_loop(0, _WIN // 2, pair, 0)
    drain_scatters(0)
    drain_scatters(1)


def _body(edges_a, vals_a, edges_d, vals_d, xcat,
          mean_out, cond_out, h1_out,
          acc, ebuf, vbuf, gath, sbuf,
          esem0, esem1, gsem0, gsem1, ssem0, ssem1):
    cid = lax.axis_index("c")
    sid = lax.axis_index("s")
    coff = cid * _NP
    rbase = sid * _NROW

    def fill_zero_chunk():
        @plsc.parallel_loop(0, _LANES, unroll=4)
        def zr(r):
            for k in range(_DH // 16):
                sbuf[(0, 0, r, pl.ds(k * 16, 16))] = jnp.zeros((16,),
                                                               jnp.float32)

    def zero_slice(dst):
        for t in range(_NROW // _LANES):
            pltpu.sync_copy(sbuf.at[0, 0],
                            dst.at[pl.ds(rbase + t * _LANES, _LANES)])

    fill_zero_chunk()
    zero_slice(acc)
    plsc.subcore_barrier()

    _spmm_pass(sid, coff, edges_a, vals_a, xcat, acc,
               ebuf, vbuf, gath, sbuf, esem0, esem1, gsem0, gsem1,
               ssem0, ssem1, True)
    plsc.subcore_barrier()
    pltpu.sync_copy(acc.at[pl.ds(rbase, _NROW)],
                    h1_out.at[pl.ds(coff + rbase, _NROW)])
    fill_zero_chunk()
    zero_slice(acc)
    plsc.subcore_barrier()

    _spmm_pass(sid, coff, edges_a, vals_a, h1_out, acc,
               ebuf, vbuf, gath, sbuf, esem0, esem1, gsem0, gsem1,
               ssem0, ssem1, True)
    plsc.subcore_barrier()

    bx, bh, b2 = gath.at[0, 0], gath.at[0, 1], sbuf.at[0, 0]
    for t in range(5):
        r0 = rbase + t * 128
        pltpu.sync_copy(xcat.at[pl.ds(coff + r0, 128)], bx)
        pltpu.sync_copy(h1_out.at[pl.ds(coff + r0, 128)], bh)
        pltpu.sync_copy(acc.at[pl.ds(r0, 128)], b2)

        @plsc.parallel_loop(0, 128, unroll=4)
        def mrow(r):
            for k in range(_DH // 16):
                sl = (r, pl.ds(k * 16, 16))
                bx[sl] = (bx[sl] + bh[sl] + b2[sl]) * jnp.float32(1.0 / 3.0)
        pltpu.sync_copy(bx, mean_out.at[pl.ds(coff + r0, 128)])
    fill_zero_chunk()
    zero_slice(acc)
    plsc.subcore_barrier()

    _spmm_pass(sid, coff, edges_d, vals_d, mean_out, acc,
               ebuf, vbuf, gath, sbuf, esem0, esem1, gsem0, gsem1,
               ssem0, ssem1, True)
    plsc.subcore_barrier()
    pltpu.sync_copy(acc.at[pl.ds(rbase, _NROW)],
                    cond_out.at[pl.ds(coff + rbase, _NROW)])


def kernel(adj_index, adj_vals, d1_index, d1_vals, uEmbeds, iEmbeds):
    embeds = jnp.concatenate([uEmbeds, iEmbeds], axis=0)
    zpad = jnp.zeros((_NP - _N, _DH), jnp.float32)
    xcat = jnp.concatenate(
        [embeds[:, :_DH], zpad, embeds[:, _DH:], zpad], axis=0)
    edges_a, vals_a = _pack_edges(adj_index, adj_vals)
    edges_d, vals_d = _pack_edges(d1_index, d1_vals)

    mesh = plsc.VectorSubcoreMesh(core_axis_name="c", subcore_axis_name="s",
                                  num_cores=2, num_subcores=16)
    f32 = jnp.float32
    half = jax.ShapeDtypeStruct((2 * _NP, _DH), f32)
    call = pl.kernel(
        _body,
        out_type=(half, half, half),
        mesh=mesh,
        compiler_params=pltpu.CompilerParams(use_tc_tiling_on_sc=False),
        scratch_types=[
            pltpu.VMEM_SHARED((_NP, _DH), f32),
            pltpu.VMEM((4, _RPW, 2, _LANES), jnp.int32),
            pltpu.VMEM((4, _RPW * _LANES), f32),
            pltpu.VMEM((2, _RPW, _LANES, _DH), f32),
            pltpu.VMEM((2, _RPW, _LANES, _DH), f32),
            pltpu.SemaphoreType.DMA,
            pltpu.SemaphoreType.DMA,
            pltpu.SemaphoreType.DMA,
            pltpu.SemaphoreType.DMA,
            pltpu.SemaphoreType.DMA,
            pltpu.SemaphoreType.DMA,
        ],
    )
    mean_h, cond_h, _ = call(edges_a, vals_a, edges_d, vals_d, xcat)
    mean = jnp.concatenate([mean_h[:_N], mean_h[_NP:_NP + _N]], axis=1)
    cond = jnp.concatenate([cond_h[:_N], cond_h[_NP:_NP + _N]], axis=1)
    return mean[:_USER], mean[_USER:], cond, uEmbeds, iEmbeds
